# baseline (speedup 1.0000x reference)
import functools

import jax
import jax.numpy as jnp
from jax import lax
from jax.experimental import pallas as pl
from jax.experimental.pallas import tpu as pltpu
from jax.experimental.pallas import tpu_sc as plsc

_NC = 1
_NS = 16
_NW = _NC * _NS
_L = 16


@functools.lru_cache(maxsize=None)
def _make_gather(batch: int, table_padded: int):
    assert batch % (_NW * _L) == 0
    b_per_w = batch // _NW

    mesh = plsc.VectorSubcoreMesh(
        core_axis_name="c", subcore_axis_name="s", num_cores=1
    )

    @functools.partial(
        pl.kernel,
        out_type=jax.ShapeDtypeStruct((batch,), jnp.float32),
        mesh=mesh,
        scratch_types=[
            pltpu.VMEM((table_padded,), jnp.float32),
            pltpu.VMEM((b_per_w,), jnp.int32),
            pltpu.VMEM((b_per_w,), jnp.float32),
            pltpu.SemaphoreType.DMA,
            pltpu.SemaphoreType.DMA,
            pltpu.SemaphoreType.DMA,
        ],
        compiler_params=pltpu.CompilerParams(needs_layout_passes=False),
    )
    def gather_kernel(
        values_hbm, t_hbm, out_hbm, tab_v, idx_v, out_v, sem_t, sem_i, sem_o
    ):
        wid = lax.axis_index("s") * _NC + lax.axis_index("c")
        base = wid * b_per_w
        copy_tab = pltpu.async_copy(values_hbm, tab_v, sem_t)
        copy_idx = pltpu.async_copy(t_hbm.at[pl.ds(base, b_per_w)], idx_v, sem_i)
        copy_tab.wait()
        copy_idx.wait()
        def step(i, carry):
            off = i * _L
            idx16 = idx_v[pl.ds(off, _L)]
            out_v[pl.ds(off, _L)] = plsc.load_gather(tab_v, [idx16])
            return carry

        lax.fori_loop(0, b_per_w // _L, step, 0, unroll=4)
        pltpu.async_copy(out_v, out_hbm.at[pl.ds(base, b_per_w)], sem_o).wait()

    return gather_kernel


def kernel(values, t, shape):
    batch = t.shape[0]
    ndim = shape.shape[0]
    out = _make_gather(batch, values.shape[0])(values, t)
    return out.reshape((batch,) + (1,) * (ndim - 1))


# speedup vs baseline: 1.0035x; 1.0035x over previous
import functools

import jax
import jax.numpy as jnp
from jax import lax
from jax.experimental import pallas as pl
from jax.experimental.pallas import tpu as pltpu
from jax.experimental.pallas import tpu_sc as plsc

_NC = 1
_NS = 16
_NW = _NC * _NS
_L = 16


@functools.lru_cache(maxsize=None)
def _make_gather(batch: int, table_padded: int):
    assert batch % (_NW * _L) == 0
    b_per_w = batch // _NW

    mesh = plsc.VectorSubcoreMesh(
        core_axis_name="c", subcore_axis_name="s", num_cores=1
    )

    @functools.partial(
        pl.kernel,
        out_type=jax.ShapeDtypeStruct((batch,), jnp.float32),
        mesh=mesh,
        scratch_types=[
            pltpu.VMEM((table_padded,), jnp.float32),
            pltpu.VMEM((b_per_w,), jnp.int32),
            pltpu.VMEM((b_per_w,), jnp.float32),
            pltpu.SemaphoreType.DMA,
            pltpu.SemaphoreType.DMA,
            pltpu.SemaphoreType.DMA,
        ],
        compiler_params=pltpu.CompilerParams(needs_layout_passes=False),
    )
    def gather_kernel(
        values_hbm, t_hbm, out_hbm, tab_v, idx_v, out_v, sem_t, sem_i, sem_o
    ):
        wid = lax.axis_index("s") * _NC + lax.axis_index("c")
        base = wid * b_per_w
        copy_tab = pltpu.async_copy(values_hbm, tab_v, sem_t)
        copy_idx = pltpu.async_copy(t_hbm.at[pl.ds(base, b_per_w)], idx_v, sem_i)
        copy_tab.wait()
        copy_idx.wait()
        n_chunks = 4
        chunk = b_per_w // n_chunks
        out_copies = []
        for c in range(n_chunks):
            for i in range(chunk // _L):
                off = c * chunk + i * _L
                idx16 = idx_v[pl.ds(off, _L)]
                out_v[pl.ds(off, _L)] = plsc.load_gather(tab_v, [idx16])
            out_copies.append(
                pltpu.async_copy(
                    out_v.at[pl.ds(c * chunk, chunk)],
                    out_hbm.at[pl.ds(base + c * chunk, chunk)],
                    sem_o,
                )
            )
        for copy in out_copies:
            copy.wait()

    return gather_kernel


def kernel(values, t, shape):
    batch = t.shape[0]
    ndim = shape.shape[0]
    out = _make_gather(batch, values.shape[0])(values, t)
    return out.reshape((batch,) + (1,) * (ndim - 1))


# speedup vs baseline: 1.0236x; 1.0201x over previous
import functools

import jax
import jax.numpy as jnp
from jax import lax
from jax.experimental import pallas as pl
from jax.experimental.pallas import tpu as pltpu
from jax.experimental.pallas import tpu_sc as plsc

_NC = 1
_NS = 16
_NW = _NC * _NS
_L = 16


@functools.lru_cache(maxsize=None)
def _make_gather(batch: int, table_padded: int):
    assert batch % (_NW * _L) == 0
    b_per_w = batch // _NW

    mesh = plsc.VectorSubcoreMesh(
        core_axis_name="c", subcore_axis_name="s", num_cores=1
    )

    @functools.partial(
        pl.kernel,
        out_type=jax.ShapeDtypeStruct((batch,), jnp.float32),
        mesh=mesh,
        scratch_types=[
            pltpu.VMEM((table_padded,), jnp.float32),
            pltpu.VMEM((b_per_w,), jnp.int32),
            pltpu.VMEM((b_per_w,), jnp.float32),
            pltpu.SemaphoreType.DMA,
            pltpu.SemaphoreType.DMA,
            pltpu.SemaphoreType.DMA,
        ],
        compiler_params=pltpu.CompilerParams(needs_layout_passes=False),
    )
    def gather_kernel(
        values_hbm, t_hbm, out_hbm, tab_v, idx_v, out_v, sem_t, sem_i, sem_o
    ):
        wid = lax.axis_index("s") * _NC + lax.axis_index("c")
        base = wid * b_per_w
        copy_tab = pltpu.async_copy(values_hbm, tab_v, sem_t)
        copy_idx = pltpu.async_copy(t_hbm.at[pl.ds(base, b_per_w)], idx_v, sem_i)
        copy_tab.wait()
        copy_idx.wait()
        @plsc.parallel_loop(0, b_per_w, step=_L, unroll=8)
        def _gather(off):
            idx16 = idx_v[pl.ds(off, _L)]
            out_v[pl.ds(off, _L)] = plsc.load_gather(tab_v, [idx16])

        pltpu.async_copy(out_v, out_hbm.at[pl.ds(base, b_per_w)], sem_o).wait()

    return gather_kernel


def kernel(values, t, shape):
    batch = t.shape[0]
    ndim = shape.shape[0]
    out = _make_gather(batch, values.shape[0])(values, t)
    return out.reshape((batch,) + (1,) * (ndim - 1))
